# baseline (speedup 1.0000x reference)
import functools

import jax
import jax.numpy as jnp
from jax.experimental import pallas as pl
from jax.experimental.pallas import tpu as pltpu

D = 1024
F = 4096
E = 8
SEQ = 2048
T = 128
MAX_T = SEQ // T + E
PAD = MAX_T * T
FT = 1024
NF = F // FT


def _gate_body(logits_ref, sel_ref):
    logits = logits_ref[...]
    m = jnp.max(logits, axis=0, keepdims=True)
    lse = m + jnp.log(jnp.sum(jnp.exp(logits - m), axis=0, keepdims=True))
    adj = logits - lse
    sel_ref[...] = jnp.argmax(adj, axis=1).astype(jnp.int32)[None, :]


def _gelu(h):
    return 0.5 * h * (1.0 + jax.lax.erf(h * 0.7071067811865476))


def _ffn_body(te_ref, nv_ref, xs_ref, w1_ref, b1_ref, w2_ref, b2_ref, out_ref):
    f = pl.program_id(0)
    t = pl.program_id(1)

    @pl.when(t < nv_ref[0])
    def _():
        h = jax.lax.dot_general(
            xs_ref[pl.ds(t * T, T), :], w1_ref[0], (((1,), (0,)), ((), ())),
            preferred_element_type=jnp.float32)
        h = _gelu(h + b1_ref[0])
        o = jax.lax.dot_general(
            h, w2_ref[0], (((1,), (0,)), ((), ())),
            preferred_element_type=jnp.float32)
        rows = pl.ds(t * T, T)

        @pl.when(f == 0)
        def _():
            out_ref[rows, :] = o + b2_ref[0]

        @pl.when(f > 0)
        def _():
            out_ref[rows, :] = out_ref[rows, :] + o


def _gate(logits):
    return pl.pallas_call(
        _gate_body,
        out_shape=jax.ShapeDtypeStruct((1, SEQ), jnp.int32),
    )(logits)[0]


def _ffn(te, nv, xs, W1, b1, W2, b2):
    grid_spec = pltpu.PrefetchScalarGridSpec(
        num_scalar_prefetch=2,
        grid=(NF, MAX_T),
        in_specs=[
            pl.BlockSpec((PAD, D), lambda f, t, te, nv: (0, 0)),
            pl.BlockSpec((1, D, FT), lambda f, t, te, nv: (te[t], 0, f)),
            pl.BlockSpec((1, 1, FT), lambda f, t, te, nv: (te[t], 0, f)),
            pl.BlockSpec((1, FT, D), lambda f, t, te, nv: (te[t], f, 0)),
            pl.BlockSpec((1, 1, D), lambda f, t, te, nv: (te[t], 0, 0)),
        ],
        out_specs=pl.BlockSpec((PAD, D), lambda f, t, te, nv: (0, 0)),
    )
    return pl.pallas_call(
        _ffn_body,
        grid_spec=grid_spec,
        out_shape=jax.ShapeDtypeStruct((PAD, D), jnp.float32),
    )(te, nv, xs, W1, b1[:, None, :], W2, b2[:, None, :])


def kernel(x, Wg, bg, W1, b1, W2, b2):
    x2 = x[0]
    logits = jnp.einsum('bld,de->ble', x, Wg) + bg
    sel = _gate(logits[0])

    onehot = (sel[:, None] == jnp.arange(E, dtype=jnp.int32)[None, :])
    inc = jnp.cumsum(onehot.astype(jnp.int32), axis=0)
    counts = inc[-1]
    rank = jnp.sum(jnp.where(onehot, inc, 0), axis=1) - 1
    n_tiles = (counts + T - 1) // T
    csum = jnp.cumsum(n_tiles)
    pt_off = csum - n_tiles
    total_tiles = csum[-1]
    p = pt_off[sel] * T + rank

    tt = jnp.arange(MAX_T, dtype=jnp.int32)
    te_raw = jnp.searchsorted(csum, tt, side='right').astype(jnp.int32)
    last_e = jnp.searchsorted(csum, total_tiles - 1, side='right').astype(jnp.int32)
    te = jnp.where(tt < total_tiles, jnp.clip(te_raw, 0, E - 1), last_e)
    nv = total_tiles.astype(jnp.int32)[None]

    xs = jnp.zeros((PAD, D), jnp.float32).at[p, :].set(x2)
    ys = _ffn(te, nv, xs, W1, b1, W2, b2)
    out = jnp.take(ys, p, axis=0)
    return out[None]


# speedup vs baseline: 1.6729x; 1.6729x over previous
import functools

import jax
import jax.numpy as jnp
from jax import lax
from jax.experimental import pallas as pl
from jax.experimental.pallas import tpu as pltpu
from jax.experimental.pallas import tpu_sc as plsc

D = 1024
F = 4096
E = 8
SEQ = 2048
T = 128
MAX_T = SEQ // T + E
PAD = MAX_T * T
FT = 1024
NF = F // FT


def _gate_body(logits_ref, p_ref, base_ref, ntil_ref):
    logits = logits_ref[...]
    m = jnp.max(logits, axis=0, keepdims=True)
    lse = m + jnp.log(jnp.sum(jnp.exp(logits - m), axis=0, keepdims=True))
    adj = logits - lse
    sel = jnp.argmax(adj, axis=1).astype(jnp.int32)

    oh = jax.lax.broadcasted_iota(jnp.int32, (SEQ, E), 1) == sel[:, None]
    inc = oh.astype(jnp.int32)
    k = 1
    while k < SEQ:
        inc = inc + jnp.concatenate(
            [jnp.zeros((k, E), jnp.int32), inc[: SEQ - k]], axis=0)
        k *= 2
    counts = inc[-1:, :]
    rank = jnp.sum(jnp.where(oh, inc, 0), axis=1) - 1

    n_tiles = (counts + T - 1) // T
    csum = n_tiles
    k = 1
    while k < E:
        csum = csum + jnp.concatenate(
            [jnp.zeros((1, k), jnp.int32), csum[:, : E - k]], axis=1)
        k *= 2
    pt_off = csum - n_tiles
    total = csum[:, -1:]

    ptok = jnp.sum(jnp.where(oh, pt_off, 0), axis=1)
    p_ref[...] = (ptok * T + rank)[None, :]
    base_ref[...] = pt_off
    ntil_ref[...] = n_tiles


def _gelu(h):
    return 0.5 * h * (1.0 + jax.lax.erf(h * 0.7071067811865476))


def _ffn_body(base_ref, ntil_ref, xs_ref, w1_ref, b1_ref, w2_ref, b2_ref, out_ref):
    e = pl.program_id(0)
    f = pl.program_id(1)
    base = base_ref[e]

    def tile_step(k, carry):
        rows = pl.ds((base + k) * T, T)
        h = jax.lax.dot_general(
            xs_ref[rows, :], w1_ref[0], (((1,), (0,)), ((), ())),
            preferred_element_type=jnp.float32)
        h = _gelu(h + b1_ref[0])
        o = jax.lax.dot_general(
            h, w2_ref[0], (((1,), (0,)), ((), ())),
            preferred_element_type=jnp.float32)

        @pl.when(f == 0)
        def _():
            out_ref[rows, :] = o + b2_ref[0]

        @pl.when(f > 0)
        def _():
            out_ref[rows, :] = out_ref[rows, :] + o

        return carry

    jax.lax.fori_loop(0, ntil_ref[e], tile_step, 0)


NW = 32
BPW = SEQ // NW


def _sc_permute_body(src_hbm, p_hbm, out_hbm, idx_v, rows_v, sem, *, scatter):
    wid = lax.axis_index("s") * 2 + lax.axis_index("c")
    base = wid * BPW
    pltpu.sync_copy(p_hbm.at[pl.ds(base, BPW)], idx_v)
    if scatter:
        pltpu.sync_copy(src_hbm.at[pl.ds(base, BPW)], rows_v)
        pltpu.async_copy(rows_v, out_hbm.at[idx_v], sem).wait()
    else:
        pltpu.async_copy(src_hbm.at[idx_v], rows_v, sem).wait()
        pltpu.sync_copy(rows_v, out_hbm.at[pl.ds(base, BPW)])


def _sc_permute(src, p, out_rows, scatter):
    body = functools.partial(_sc_permute_body, scatter=scatter)
    return pl.kernel(
        body,
        out_type=jax.ShapeDtypeStruct((out_rows, D), jnp.float32),
        mesh=plsc.VectorSubcoreMesh(core_axis_name="c", subcore_axis_name="s"),
        scratch_types=[
            pltpu.VMEM((BPW,), jnp.int32),
            pltpu.VMEM((BPW, D), jnp.float32),
            pltpu.SemaphoreType.DMA,
        ],
    )(src, p)


def _gate(logits):
    p, base, ntil = pl.pallas_call(
        _gate_body,
        out_shape=(
            jax.ShapeDtypeStruct((1, SEQ), jnp.int32),
            jax.ShapeDtypeStruct((1, E), jnp.int32),
            jax.ShapeDtypeStruct((1, E), jnp.int32),
        ),
    )(logits)
    return p[0], base[0], ntil[0]


def _ffn(base, ntil, xs, W1, b1, W2, b2):
    grid_spec = pltpu.PrefetchScalarGridSpec(
        num_scalar_prefetch=2,
        grid=(E, NF),
        in_specs=[
            pl.BlockSpec((PAD, D), lambda e, f, base, ntil: (0, 0)),
            pl.BlockSpec((1, D, FT), lambda e, f, base, ntil: (e, 0, f)),
            pl.BlockSpec((1, 1, FT), lambda e, f, base, ntil: (e, 0, f)),
            pl.BlockSpec((1, FT, D), lambda e, f, base, ntil: (e, f, 0)),
            pl.BlockSpec((1, 1, D), lambda e, f, base, ntil: (e, 0, 0)),
        ],
        out_specs=pl.BlockSpec((PAD, D), lambda e, f, base, ntil: (0, 0)),
    )
    return pl.pallas_call(
        _ffn_body,
        grid_spec=grid_spec,
        out_shape=jax.ShapeDtypeStruct((PAD, D), jnp.float32),
    )(base, ntil, xs, W1, b1[:, None, :], W2, b2[:, None, :])


def kernel(x, Wg, bg, W1, b1, W2, b2):
    x2 = x[0]
    logits = jnp.einsum('bld,de->ble', x, Wg) + bg
    p, base, ntil = _gate(logits[0])

    xs = _sc_permute(x2, p, PAD, scatter=True)
    ys = _ffn(base, ntil, xs, W1, b1, W2, b2)
    out = _sc_permute(ys, p, SEQ, scatter=False)
    return out[None]


# speedup vs baseline: 1.7678x; 1.0567x over previous
import functools

import jax
import jax.numpy as jnp
from jax import lax
from jax.experimental import pallas as pl
from jax.experimental.pallas import tpu as pltpu
from jax.experimental.pallas import tpu_sc as plsc

D = 1024
F = 4096
E = 8
SEQ = 2048
T = 128
MAX_T = SEQ // T + E
PAD = MAX_T * T
FT = 2048
NF = F // FT


def _gate_body(logits_ref, p_ref, base_ref, ntil_ref):
    logits = logits_ref[...]
    m = jnp.max(logits, axis=0, keepdims=True)
    lse = m + jnp.log(jnp.sum(jnp.exp(logits - m), axis=0, keepdims=True))
    adj = logits - lse
    sel = jnp.argmax(adj, axis=1).astype(jnp.int32)

    oh = jax.lax.broadcasted_iota(jnp.int32, (SEQ, E), 1) == sel[:, None]
    inc = oh.astype(jnp.int32)
    k = 1
    while k < SEQ:
        inc = inc + jnp.concatenate(
            [jnp.zeros((k, E), jnp.int32), inc[: SEQ - k]], axis=0)
        k *= 2
    counts = inc[-1:, :]
    rank = jnp.sum(jnp.where(oh, inc, 0), axis=1) - 1

    n_tiles = (counts + T - 1) // T
    csum = n_tiles
    k = 1
    while k < E:
        csum = csum + jnp.concatenate(
            [jnp.zeros((1, k), jnp.int32), csum[:, : E - k]], axis=1)
        k *= 2
    pt_off = csum - n_tiles
    total = csum[:, -1:]

    ptok = jnp.sum(jnp.where(oh, pt_off, 0), axis=1)
    p_ref[...] = (ptok * T + rank)[None, :]
    base_ref[...] = pt_off
    ntil_ref[...] = n_tiles


def _gelu(h):
    return 0.5 * h * (1.0 + jax.lax.erf(h * 0.7071067811865476))


def _ffn_body(base_ref, ntil_ref, xs_ref, w1_ref, b1_ref, w2_ref, b2_ref, out_ref):
    e = pl.program_id(0)
    f = pl.program_id(1)
    base = base_ref[e]

    def tile_step(k, carry):
        rows = pl.ds((base + k) * T, T)
        h = jax.lax.dot_general(
            xs_ref[rows, :], w1_ref[0], (((1,), (0,)), ((), ())),
            preferred_element_type=jnp.float32)
        h = _gelu(h + b1_ref[0])
        o = jax.lax.dot_general(
            h, w2_ref[0], (((1,), (0,)), ((), ())),
            preferred_element_type=jnp.float32)

        @pl.when(f == 0)
        def _():
            out_ref[rows, :] = o + b2_ref[0]

        @pl.when(f > 0)
        def _():
            out_ref[rows, :] = out_ref[rows, :] + o

        return carry

    jax.lax.fori_loop(0, ntil_ref[e], tile_step, 0)


NW = 32
BPW = SEQ // NW


def _sc_permute_body(src_hbm, p_hbm, out_hbm, idx_v, rows_v, sem, *, scatter):
    wid = lax.axis_index("s") * 2 + lax.axis_index("c")
    base = wid * BPW
    pltpu.sync_copy(p_hbm.at[pl.ds(base, BPW)], idx_v)
    if scatter:
        pltpu.sync_copy(src_hbm.at[pl.ds(base, BPW)], rows_v)
        pltpu.async_copy(rows_v, out_hbm.at[idx_v], sem).wait()
    else:
        pltpu.async_copy(src_hbm.at[idx_v], rows_v, sem).wait()
        pltpu.sync_copy(rows_v, out_hbm.at[pl.ds(base, BPW)])


def _sc_permute(src, p, out_rows, scatter):
    body = functools.partial(_sc_permute_body, scatter=scatter)
    return pl.kernel(
        body,
        out_type=jax.ShapeDtypeStruct((out_rows, D), jnp.float32),
        mesh=plsc.VectorSubcoreMesh(core_axis_name="c", subcore_axis_name="s"),
        scratch_types=[
            pltpu.VMEM((BPW,), jnp.int32),
            pltpu.VMEM((BPW, D), jnp.float32),
            pltpu.SemaphoreType.DMA,
        ],
    )(src, p)


def _gate(logits):
    p, base, ntil = pl.pallas_call(
        _gate_body,
        out_shape=(
            jax.ShapeDtypeStruct((1, SEQ), jnp.int32),
            jax.ShapeDtypeStruct((1, E), jnp.int32),
            jax.ShapeDtypeStruct((1, E), jnp.int32),
        ),
    )(logits)
    return p[0], base[0], ntil[0]


def _ffn(base, ntil, xs, W1, b1, W2, b2):
    grid_spec = pltpu.PrefetchScalarGridSpec(
        num_scalar_prefetch=2,
        grid=(E, NF),
        in_specs=[
            pl.BlockSpec((PAD, D), lambda e, f, base, ntil: (0, 0)),
            pl.BlockSpec((1, D, FT), lambda e, f, base, ntil: (e, 0, f)),
            pl.BlockSpec((1, 1, FT), lambda e, f, base, ntil: (e, 0, f)),
            pl.BlockSpec((1, FT, D), lambda e, f, base, ntil: (e, f, 0)),
            pl.BlockSpec((1, 1, D), lambda e, f, base, ntil: (e, 0, 0)),
        ],
        out_specs=pl.BlockSpec((PAD, D), lambda e, f, base, ntil: (0, 0)),
    )
    return pl.pallas_call(
        _ffn_body,
        grid_spec=grid_spec,
        out_shape=jax.ShapeDtypeStruct((PAD, D), jnp.float32),
    )(base, ntil, xs, W1, b1[:, None, :], W2, b2[:, None, :])


def kernel(x, Wg, bg, W1, b1, W2, b2):
    x2 = x[0]
    logits = jnp.einsum('bld,de->ble', x, Wg) + bg
    p, base, ntil = _gate(logits[0])

    xs = _sc_permute(x2, p, PAD, scatter=True)
    ys = _ffn(base, ntil, xs, W1, b1, W2, b2)
    out = _sc_permute(ys, p, SEQ, scatter=False)
    return out[None]
